# baseline (speedup 1.0000x reference)
import functools

import jax
import jax.numpy as jnp
from jax import lax
from jax.experimental import pallas as pl
from jax.experimental.pallas import tpu as pltpu
from jax.experimental.pallas import tpu_sc as plsc

_RB = 8
_GW = 128
_NREF = 4
_SC_ROWS = 64
_SC_COLS = 99840
_CW = 4992
_QW = 24960
_TAILW = 256
_L = 16


def _gather_body(tstart_ref, p_ref, tmod_ref, v_ref, vmat_ref, x_scr, sem,
                 *, bsz, bszt):
    copies = [
        pltpu.make_async_copy(
            p_ref.at[pl.ds(8 * (r // 8), 8),
                     pl.ds(pl.multiple_of(tstart_ref[r], _GW), _GW)],
            x_scr.at[r],
            sem,
        )
        for r in range(bsz)
    ]
    for c in copies:
        c.start()
    for c in copies:
        c.wait()
    sub = jax.lax.broadcasted_iota(jnp.int32, (bsz, 8, _GW), 1)
    rmod = jax.lax.broadcasted_iota(jnp.int32, (bsz, 8, _GW), 0) % 8
    lane = jax.lax.broadcasted_iota(jnp.int32, (bsz, 8, _GW), 2)
    sel = jnp.where((sub == rmod) & (lane == tmod_ref[...]), x_scr[...], 0.0)
    v = jnp.sum(sel, axis=(1, 2)).reshape(bsz, 1)
    v_ref[...] = v
    vmat_ref[...] = jnp.broadcast_to(v[bszt:], (bsz - bszt, _GW))


def _tc_body(p_ref, v_ref, t_ref, *x_refs_and_outs, nsteps, bszt, bsz, n):
    x_refs = x_refs_and_outs[:_NREF]
    rank_ref, rtail_ref = x_refs_and_outs[_NREF:_NREF + 2]
    xt_scr, sem = x_refs_and_outs[_NREF + 2:]
    j = pl.program_id(0)

    for r, xr in enumerate(x_refs):
        g = j * _NREF + r
        x = xr[...]
        v = v_ref[pl.ds(g * _RB, _RB), :]
        t = t_ref[pl.ds(g * _RB, _RB), :]
        lane = jax.lax.broadcasted_iota(jnp.int32, x.shape, 1)
        beat = (x > v) | ((x == v) & (lane < t))
        rank_ref[pl.ds(g * _RB, _RB), :] = jnp.sum(beat.astype(jnp.int32),
                                                   axis=1, keepdims=True)

    @pl.when(j == nsteps - 1)
    def _tail():
        nsc = bsz - bszt
        cp = pltpu.make_async_copy(
            p_ref.at[pl.ds(bszt, nsc),
                     pl.ds(pl.multiple_of(0 * j + _SC_COLS, _GW), _TAILW)],
            xt_scr, sem)
        cp.start()
        cp.wait()
        xt = xt_scr[...]
        vt = v_ref[pl.ds(bszt, nsc), :]
        tt = t_ref[pl.ds(bszt, nsc), :]
        lane = _SC_COLS + jax.lax.broadcasted_iota(jnp.int32, xt.shape, 1)
        beat = ((xt > vt) | ((xt == vt) & (lane < tt))) & (lane < n)
        rtail_ref[...] = jnp.sum(beat.astype(jnp.int32), axis=1,
                                 keepdims=True)


def _merge_body(rtc_ref, rtail_ref, rsc_ref, out1_ref, out5_ref, *, bsz):
    r1 = rtc_ref[...]
    x = rsc_ref[...]
    r2 = jnp.sum(x[0:8] + x[8:16] + x[16:24] + x[24:32], axis=2)
    r2 = r2 + rtail_ref[...].reshape(8, 8)
    scale = 100.0 / bsz
    out1_ref[...] = ((jnp.sum((r1 < 1).astype(jnp.float32), axis=(0, 1),
                              keepdims=True)
                      + jnp.sum((r2 < 1).astype(jnp.float32), axis=(0, 1),
                                keepdims=True)) * scale)
    out5_ref[...] = ((jnp.sum((r1 < 5).astype(jnp.float32), axis=(0, 1),
                              keepdims=True)
                      + jnp.sum((r2 < 5).astype(jnp.float32), axis=(0, 1),
                                keepdims=True)) * scale)


def kernel(preds, targets):
    bsz, n = preds.shape
    bszt = bsz - _SC_ROWS
    t32 = targets.astype(jnp.int32)
    tstart = (t32 // _GW) * _GW
    tmod = (t32 % _GW).reshape(bsz, 1, 1)
    tmat = jnp.broadcast_to(t32[bszt:].reshape(_SC_ROWS, 1),
                            (_SC_ROWS, _GW))

    info = plsc.get_sparse_core_info()
    nc = info.num_cores

    v, vmat = pl.pallas_call(
        functools.partial(_gather_body, bsz=bsz, bszt=bszt),
        grid_spec=pltpu.PrefetchScalarGridSpec(
            num_scalar_prefetch=1,
            grid=(1,),
            in_specs=[
                pl.BlockSpec(memory_space=pl.ANY),
                pl.BlockSpec((bsz, 1, 1), lambda j, s: (0, 0, 0)),
            ],
            out_specs=[
                pl.BlockSpec((bsz, 1), lambda j, s: (0, 0)),
                pl.BlockSpec((_SC_ROWS, _GW), lambda j, s: (0, 0)),
            ],
            scratch_shapes=[
                pltpu.VMEM((bsz, 8, _GW), jnp.float32),
                pltpu.SemaphoreType.DMA,
            ],
        ),
        out_shape=[jax.ShapeDtypeStruct((bsz, 1), jnp.float32),
                   jax.ShapeDtypeStruct((_SC_ROWS, _GW), jnp.float32)],
    )(tstart, preds, tmod)

    @functools.partial(
        pl.kernel,
        out_type=jax.ShapeDtypeStruct((32, 8, _L), jnp.int32),
        mesh=plsc.VectorSubcoreMesh(core_axis_name="c", subcore_axis_name="s"),
        scratch_types=[
            pltpu.VMEM((8, _GW), jnp.float32),
            pltpu.VMEM((8, _GW), jnp.int32),
            pltpu.VMEM((8, _CW), jnp.float32),
            pltpu.VMEM((8, _CW), jnp.float32),
            pltpu.VMEM((_L,), jnp.int32),
            pltpu.SemaphoreType.DMA,
            pltpu.SemaphoreType.DMA,
            pltpu.SemaphoreType.DMA,
        ],
    )
    def _sc_count(p_ref, vmat_ref, tmat_ref, out_ref, vbufv, tbufv,
                  buf0, buf1, cnt_scr, sem0, sem1, semt):
        wid = lax.axis_index("s") * nc + lax.axis_index("c")
        s = wid % 8
        q = wid // 8
        base8 = pl.multiple_of(bszt + 8 * s, 8)
        base8v = pl.multiple_of(8 * s, 8)
        qb = pl.multiple_of(q * _QW, _GW)
        lanei = lax.iota(jnp.int32, _L)
        bufs = (buf0, buf1)
        sems = (sem0, sem1)
        pltpu.async_copy(vmat_ref.at[pl.ds(base8v, 8), :], vbufv, semt).wait()
        pltpu.async_copy(tmat_ref.at[pl.ds(base8v, 8), :], tbufv, semt).wait()
        vrs = [vbufv[ri, pl.ds(0, _L)] for ri in range(8)]
        trs = [tbufv[ri, pl.ds(0, _L)] for ri in range(8)]

        nch = _QW // _CW

        def _mk(k):
            return pltpu.make_async_copy(
                p_ref.at[pl.ds(base8, 8),
                         pl.ds(pl.multiple_of(qb + k * _CW, _GW), _CW)],
                bufs[k % 2], sems[k % 2])

        cps = [_mk(k) for k in range(nch)]
        cps[0].start()
        cnts = [jnp.zeros((_L,), jnp.int32) for _ in range(8)]
        for k in range(nch):
            if k + 1 < nch:
                cps[k + 1].start()
            cps[k].wait()
            buf = bufs[k % 2]

            def gbody(g, carry, buf=buf):
                cnts = list(carry[:8])
                lane = carry[8]
                for ri in range(8):
                    x16 = buf[ri, pl.ds(g * _L, _L)]
                    beat = (x16 > vrs[ri]) | ((x16 == vrs[ri])
                                              & (lane < trs[ri]))
                    cnts[ri] = cnts[ri] + jnp.where(beat, 1, 0)
                return tuple(cnts) + (lane + _L,)

            res = lax.fori_loop(0, _CW // _L, gbody,
                                tuple(cnts) + (qb + k * _CW + lanei,))
            cnts = list(res[:8])
        for ri in range(8):
            cnt_scr[...] = cnts[ri]
            pltpu.sync_copy(cnt_scr, out_ref.at[wid, ri])

    rank_sc = _sc_count(preds, vmat, tmat)

    nsteps = bszt // (_RB * _NREF)
    rank_tc, rank_tail = pl.pallas_call(
        functools.partial(_tc_body, nsteps=nsteps, bszt=bszt, bsz=bsz, n=n),
        grid=(nsteps,),
        in_specs=[
            pl.BlockSpec(memory_space=pl.ANY),
            pl.BlockSpec((bsz, 1), lambda j: (0, 0)),
            pl.BlockSpec((bsz, 1), lambda j: (0, 0)),
        ] + [
            pl.BlockSpec((_RB, n), lambda j, r=r: (j * _NREF + r, 0))
            for r in range(_NREF)
        ],
        out_specs=[
            pl.BlockSpec((bszt, 1), lambda j: (0, 0)),
            pl.BlockSpec((_SC_ROWS, 1), lambda j: (0, 0)),
        ],
        scratch_shapes=[
            pltpu.VMEM((_SC_ROWS, _TAILW), jnp.float32),
            pltpu.SemaphoreType.DMA,
        ],
        out_shape=[jax.ShapeDtypeStruct((bszt, 1), jnp.int32),
                   jax.ShapeDtypeStruct((_SC_ROWS, 1), jnp.int32)],
    )(preds, v, t32.reshape(bsz, 1), *([preds] * _NREF))

    out1, out5 = pl.pallas_call(
        functools.partial(_merge_body, bsz=bsz),
        in_specs=[
            pl.BlockSpec((bszt, 1), lambda: (0, 0)),
            pl.BlockSpec((_SC_ROWS, 1), lambda: (0, 0)),
            pl.BlockSpec((32, 8, _L), lambda: (0, 0, 0)),
        ],
        out_specs=[
            pl.BlockSpec((1, 1), lambda: (0, 0)),
            pl.BlockSpec((1, 1), lambda: (0, 0)),
        ],
        out_shape=[jax.ShapeDtypeStruct((1, 1), jnp.float32)] * 2,
    )(rank_tc, rank_tail, rank_sc)

    return (out1.reshape(1), out5.reshape(1))


# speedup vs baseline: 1.3306x; 1.3306x over previous
import functools

import jax
import jax.numpy as jnp
from jax.experimental import pallas as pl
from jax.experimental.pallas import tpu as pltpu

_RB = 8
_GW = 128
_NREF = 4


def _body(tstart_ref, p_ref, *refs, nsteps, bsz):
    (x_refs, (tmod_ref, t_ref), (out1_ref, out5_ref),
     (v_scr, x_scr, rank_scr, sem)) = (refs[:_NREF], refs[_NREF:_NREF + 2],
                                       refs[_NREF + 2:_NREF + 4],
                                       refs[_NREF + 4:])
    j = pl.program_id(0)

    @pl.when(j == 0)
    def _gather():
        copies = [
            pltpu.make_async_copy(
                p_ref.at[pl.ds(8 * (r // 8), 8),
                         pl.ds(pl.multiple_of(tstart_ref[r], _GW), _GW)],
                x_scr.at[r],
                sem,
            )
            for r in range(bsz)
        ]
        for c in copies:
            c.start()
        for c in copies:
            c.wait()
        sub = jax.lax.broadcasted_iota(jnp.int32, (bsz, 8, _GW), 1)
        rmod = jax.lax.broadcasted_iota(jnp.int32, (bsz, 8, _GW), 0) % 8
        lane = jax.lax.broadcasted_iota(jnp.int32, (bsz, 8, _GW), 2)
        sel = jnp.where((sub == rmod) & (lane == tmod_ref[...]),
                        x_scr[...], 0.0)
        v_scr[...] = jnp.sum(sel, axis=(1, 2)).reshape(bsz, 1)

    for r, xr in enumerate(x_refs):
        g = j * _NREF + r
        x = xr[...]
        v = v_scr[pl.ds(g * _RB, _RB), :]
        t = t_ref[pl.ds(g * _RB, _RB), :]
        lane = jax.lax.broadcasted_iota(jnp.int32, x.shape, 1)
        beat = (x > v) | ((x == v) & (lane < t))
        rank_scr[pl.ds(g * _RB, _RB), :] = jnp.sum(beat.astype(jnp.int32),
                                                   axis=1, keepdims=True)

    @pl.when(j == nsteps - 1)
    def _fin():
        rank = rank_scr[...]
        scale = 100.0 / bsz
        out1_ref[...] = jnp.sum((rank < 1).astype(jnp.float32),
                                axis=(0, 1), keepdims=True) * scale
        out5_ref[...] = jnp.sum((rank < 5).astype(jnp.float32),
                                axis=(0, 1), keepdims=True) * scale


def kernel(preds, targets):
    bsz, n = preds.shape
    t32 = targets.astype(jnp.int32)
    tstart = (t32 // _GW) * _GW
    tmod = (t32 % _GW).reshape(bsz, 1, 1)

    nsteps = bsz // (_RB * _NREF)
    out1, out5 = pl.pallas_call(
        functools.partial(_body, nsteps=nsteps, bsz=bsz),
        grid_spec=pltpu.PrefetchScalarGridSpec(
            num_scalar_prefetch=1,
            grid=(nsteps,),
            in_specs=[
                pl.BlockSpec(memory_space=pl.ANY),
            ] + [
                pl.BlockSpec((_RB, n), lambda j, s, r=r: (j * _NREF + r, 0))
                for r in range(_NREF)
            ] + [
                pl.BlockSpec((bsz, 1, 1), lambda j, s: (0, 0, 0)),
                pl.BlockSpec((bsz, 1), lambda j, s: (0, 0)),
            ],
            out_specs=[
                pl.BlockSpec((1, 1), lambda j, s: (0, 0)),
                pl.BlockSpec((1, 1), lambda j, s: (0, 0)),
            ],
            scratch_shapes=[
                pltpu.VMEM((bsz, 1), jnp.float32),
                pltpu.VMEM((bsz, 8, _GW), jnp.float32),
                pltpu.VMEM((bsz, 1), jnp.int32),
                pltpu.SemaphoreType.DMA,
            ],
        ),
        out_shape=[jax.ShapeDtypeStruct((1, 1), jnp.float32)] * 2,
    )(tstart, preds, *([preds] * _NREF), tmod, t32.reshape(bsz, 1))

    return (out1.reshape(1), out5.reshape(1))
